# baseline (speedup 1.0000x reference)
import jax
import jax.numpy as jnp
from jax.experimental import pallas as pl
from jax.experimental.pallas import tpu as pltpu

_BI = 400


def _fused_body(adj_ref, feat_ref, feati_ref, ws_ref, wn_ref, out_ref):
    neigh = jnp.dot(adj_ref[...], feat_ref[...],
                    preferred_element_type=jnp.float32)
    neigh = jnp.dot(neigh, wn_ref[...], preferred_element_type=jnp.float32)
    selfp = jnp.dot(feati_ref[...], ws_ref[...],
                    preferred_element_type=jnp.float32)
    out_ref[...] = jnp.maximum(neigh + selfp, 0.0)


def kernel(feat, adj, weight_self, weight_neigh):
    n, d_in = feat.shape
    d_out = weight_self.shape[1]
    bi = min(_BI, n)
    ni = n // bi
    return pl.pallas_call(
        _fused_body,
        grid=(ni,),
        in_specs=[
            pl.BlockSpec((bi, n), lambda i: (i, 0)),
            pl.BlockSpec((n, d_in), lambda i: (0, 0)),
            pl.BlockSpec((bi, d_in), lambda i: (i, 0)),
            pl.BlockSpec((d_in, d_out), lambda i: (0, 0)),
            pl.BlockSpec((d_in, d_out), lambda i: (0, 0)),
        ],
        out_specs=pl.BlockSpec((bi, d_out), lambda i: (i, 0)),
        out_shape=jax.ShapeDtypeStruct((n, d_out), jnp.float32),
        compiler_params=pltpu.CompilerParams(
            dimension_semantics=("parallel",),
            vmem_limit_bytes=110 * 1024 * 1024,
        ),
    )(adj, feat, feat, weight_self, weight_neigh)


# speedup vs baseline: 1.0405x; 1.0405x over previous
import jax
import jax.numpy as jnp
from jax.experimental import pallas as pl
from jax.experimental.pallas import tpu as pltpu

_BI = 400


def _fused_body(adj_ref, feat_ref, ws_ref, wn_ref, out_ref, *, bi):
    i = pl.program_id(0)
    neigh = jnp.dot(adj_ref[...], feat_ref[...],
                    preferred_element_type=jnp.float32)
    neigh = jnp.dot(neigh, wn_ref[...], preferred_element_type=jnp.float32)
    feati = feat_ref[pl.ds(i * bi, bi), :]
    selfp = jnp.dot(feati, ws_ref[...], preferred_element_type=jnp.float32)
    out_ref[...] = jnp.maximum(neigh + selfp, 0.0)


def kernel(feat, adj, weight_self, weight_neigh):
    n, d_in = feat.shape
    d_out = weight_self.shape[1]
    bi = min(_BI, n)
    ni = n // bi
    import functools
    return pl.pallas_call(
        functools.partial(_fused_body, bi=bi),
        grid=(ni,),
        in_specs=[
            pl.BlockSpec((bi, n), lambda i: (i, 0)),
            pl.BlockSpec((n, d_in), lambda i: (0, 0)),
            pl.BlockSpec((d_in, d_out), lambda i: (0, 0)),
            pl.BlockSpec((d_in, d_out), lambda i: (0, 0)),
        ],
        out_specs=pl.BlockSpec((bi, d_out), lambda i: (i, 0)),
        out_shape=jax.ShapeDtypeStruct((n, d_out), jnp.float32),
        compiler_params=pltpu.CompilerParams(
            dimension_semantics=("parallel",),
        ),
    )(adj, feat, weight_self, weight_neigh)
